# baseline (speedup 1.0000x reference)
import functools

import jax
import jax.numpy as jnp
from jax import lax
from jax.experimental import pallas as pl
from jax.experimental.pallas import tpu as pltpu
from jax.experimental.pallas import tpu_sc as plsc

N_NODES = 10000
D_FEAT = 128
N_EDGES = 320000

NC = 2
NS = 16
NW = NC * NS
K = 64
E_PAD = 327680
EPW = E_PAD // NW
NB = 2
CB = 80
N_PAD = 10240
RPT = N_PAD // NS


def _sc_scatter_body(x_hbm, src_hbm, dst_hbm, ew_hbm, acc_hbm,
                     acc_sh, src_v, dst_v, ew_v, rows_a, rows_b,
                     ga, gb, sa, sb):
    c = lax.axis_index("c")
    s = lax.axis_index("s")
    gid = c * NS + s

    def _zero(i, carry):
        rows_a[i // 8, pl.ds((i % 8) * 16, 16)] = jnp.zeros((16,), jnp.float32)
        return carry
    lax.fori_loop(0, K * 8, _zero, 0)
    for j in range(RPT // K):
        pltpu.sync_copy(rows_a, acc_sh.at[pl.ds(s * RPT + j * K, K)])
    plsc.subcore_barrier()

    def _scale(buf, base, g, inner):
        w_win = ew_v[pl.ds(base + g * 16, 16)]
        for r16 in range(16):
            r = g * 16 + r16
            w16 = jnp.broadcast_to(w_win[r16], (16,))
            for cc in range(8):
                sl = pl.ds(cc * 16, 16)
                buf[r, sl] = buf[r, sl] * w16
        return inner

    def _widen_scale(buf, k):
        lax.fori_loop(0, K // 16, functools.partial(_scale, buf, k * K), 0)

    def _g_issue(buf, sem, k):
        pltpu.async_copy(x_hbm.at[src_v.at[k]], buf, sem)

    def _g_wait(buf, sem, k):
        pltpu.make_async_copy(x_hbm.at[src_v.at[k]], buf, sem).wait()

    def _s_issue(buf, sem, k):
        pltpu.async_copy(buf, acc_sh.at[dst_v.at[k]], sem, add=True)

    def _s_wait(buf, sem, k):
        pltpu.make_async_copy(buf, acc_sh.at[dst_v.at[k]], sem).wait()

    def _block(b, carry):
        pltpu.sync_copy(src_hbm.at[gid, b], src_v)
        pltpu.sync_copy(dst_hbm.at[gid, b], dst_v)
        pltpu.sync_copy(ew_hbm.at[gid, b], ew_v)

        _g_issue(rows_a, ga, 0)
        _g_issue(rows_b, gb, 1)

        def _pair(p, c2):
            k0 = 2 * p
            _g_wait(rows_a, ga, k0)
            _widen_scale(rows_a, k0)
            _s_issue(rows_a, sa, k0)
            _g_wait(rows_b, gb, k0 + 1)
            _widen_scale(rows_b, k0 + 1)
            _s_issue(rows_b, sb, k0 + 1)
            _s_wait(rows_a, sa, k0)

            @pl.when(k0 + 2 < CB)
            def _():
                _g_issue(rows_a, ga, k0 + 2)
            _s_wait(rows_b, sb, k0 + 1)

            @pl.when(k0 + 3 < CB)
            def _():
                _g_issue(rows_b, gb, k0 + 3)
            return c2
        lax.fori_loop(0, CB // 2, _pair, 0)
        return carry
    lax.fori_loop(0, NB, _block, 0)

    plsc.subcore_barrier()
    pltpu.sync_copy(acc_sh.at[pl.ds(s * RPT, RPT)],
                    acc_hbm.at[c, pl.ds(s * RPT, RPT)])


_sc_scatter = functools.partial(
    pl.kernel,
    out_type=jax.ShapeDtypeStruct((NC, N_PAD, D_FEAT), jnp.float32),
    mesh=plsc.VectorSubcoreMesh(core_axis_name="c", subcore_axis_name="s"),
    scratch_types=[
        pltpu.VMEM_SHARED((N_PAD, D_FEAT), jnp.float32),
        pltpu.VMEM((CB, K), jnp.int32),
        pltpu.VMEM((CB, K), jnp.int32),
        pltpu.VMEM((CB * K,), jnp.float32),
        pltpu.VMEM((K, D_FEAT), jnp.float32),
        pltpu.VMEM((K, D_FEAT), jnp.float32),
        pltpu.SemaphoreType.DMA,
        pltpu.SemaphoreType.DMA,
        pltpu.SemaphoreType.DMA,
        pltpu.SemaphoreType.DMA,
    ],
)(_sc_scatter_body)


def _finish_body(acc_ref, o_ref):
    t = acc_ref[0] + acc_ref[1]
    t = jnp.maximum(t, 0.0)
    nrm = jnp.sqrt(jnp.sum(t * t, axis=1, keepdims=True))
    o_ref[...] = t / jnp.maximum(nrm, 1e-12)


_ROWS_PER_BLK = 1024


def _finish(acc):
    return pl.pallas_call(
        _finish_body,
        grid=(N_PAD // _ROWS_PER_BLK,),
        in_specs=[pl.BlockSpec((NC, _ROWS_PER_BLK, D_FEAT),
                               lambda i: (0, i, 0))],
        out_specs=pl.BlockSpec((_ROWS_PER_BLK, D_FEAT), lambda i: (i, 0)),
        out_shape=jax.ShapeDtypeStruct((N_PAD, D_FEAT), jnp.float32),
    )(acc)


def kernel(x, edge, edge_weight):
    npad = E_PAD - N_EDGES
    src = jnp.concatenate(
        [edge[0], jnp.zeros((npad,), jnp.int32)]).reshape(NW, NB, CB, K)
    dst = jnp.concatenate(
        [edge[2],
         N_NODES + (jnp.arange(npad, dtype=jnp.int32) % (N_PAD - N_NODES))]
    ).reshape(NW, NB, CB, K)
    ew = jnp.concatenate(
        [edge_weight, jnp.zeros((npad,), jnp.float32)]
    ).reshape(NW, NB, CB * K)
    acc = _sc_scatter(x, src, dst, ew)
    return _finish(acc)[:N_NODES]


# speedup vs baseline: 2.9051x; 2.9051x over previous
import functools

import jax
import jax.numpy as jnp
from jax import lax
from jax.experimental import pallas as pl
from jax.experimental.pallas import tpu as pltpu
from jax.experimental.pallas import tpu_sc as plsc

N_NODES = 10000
D_FEAT = 128
N_EDGES = 320000

NC = 2
NS = 16
NW = NC * NS
K = 80
EPW = N_EDGES // NW
NB = 5
CB = 25
NBUF = 3
N_PAD = 10240
RPT = N_PAD // NS


def _sc_scatter_body(x_hbm, src_hbm, dst_hbm, ew_hbm, acc_hbm,
                     acc_sh, src_v, dst_v, ew_v,
                     rows0, rows1, rows2,
                     g0, g1, g2, s0, s1, s2):
    rows = (rows0, rows1, rows2)
    gsem = (g0, g1, g2)
    ssem = (s0, s1, s2)
    c = lax.axis_index("c")
    s = lax.axis_index("s")
    gid = c * NS + s

    def _zero(i, carry):
        rows0[i // 8, pl.ds((i % 8) * 16, 16)] = jnp.zeros((16,), jnp.float32)
        return carry
    lax.fori_loop(0, K * 8, _zero, 0)
    for j in range(RPT // K):
        pltpu.sync_copy(rows0, acc_sh.at[pl.ds(s * RPT + j * K, K)])
    plsc.subcore_barrier()

    def _scale(buf, base, g, inner):
        w_win = ew_v[pl.ds(base + g * 16, 16)]
        for r16 in range(16):
            r = g * 16 + r16
            w16 = jnp.broadcast_to(w_win[r16], (16,))
            for cc in range(8):
                sl = pl.ds(cc * 16, 16)
                buf[r, sl] = buf[r, sl] * w16
        return inner

    def _widen_scale(buf, k):
        lax.fori_loop(0, K // 16, functools.partial(_scale, buf, k * K), 0)

    def _g_issue(buf, sem, k):
        pltpu.async_copy(x_hbm.at[src_v.at[k]], buf, sem)

    def _g_wait(buf, sem, k):
        pltpu.make_async_copy(x_hbm.at[src_v.at[k]], buf, sem).wait()

    def _s_issue(buf, sem, k):
        pltpu.async_copy(buf, acc_sh.at[dst_v.at[k]], sem, add=True)

    def _s_wait(buf, sem, k):
        pltpu.make_async_copy(buf, acc_sh.at[dst_v.at[k]], sem).wait()

    def _block(b, carry):
        pltpu.sync_copy(src_hbm.at[gid, b], src_v)
        pltpu.sync_copy(dst_hbm.at[gid, b], dst_v)
        pltpu.sync_copy(ew_hbm.at[gid, b], ew_v)

        for i in range(NBUF):
            _g_issue(rows[i], gsem[i], i)

        def _tri(q, c2):
            k0 = q * NBUF
            for i in range(NBUF):
                k = k0 + i
                _g_wait(rows[i], gsem[i], k)
                _widen_scale(rows[i], k)
                _s_issue(rows[i], ssem[i], k)
            for i in range(NBUF):
                k = k0 + i
                _s_wait(rows[i], ssem[i], k)
                kn = k0 + NBUF + i

                @pl.when(kn < CB)
                def _issue_next(kn=kn, i=i):
                    _g_issue(rows[i], gsem[i], kn)
            return c2
        lax.fori_loop(0, CB // NBUF, _tri, 0)

        kt = (CB // NBUF) * NBUF
        _g_wait(rows0, g0, kt)
        _widen_scale(rows0, kt)
        pltpu.sync_copy(rows0, acc_sh.at[dst_v.at[kt]], add=True)
        return carry
    lax.fori_loop(0, NB, _block, 0)

    plsc.subcore_barrier()
    pltpu.sync_copy(acc_sh.at[pl.ds(s * RPT, RPT)],
                    acc_hbm.at[c, pl.ds(s * RPT, RPT)])


_sc_scatter = functools.partial(
    pl.kernel,
    out_type=jax.ShapeDtypeStruct((NC, N_PAD, D_FEAT), jnp.float32),
    mesh=plsc.VectorSubcoreMesh(core_axis_name="c", subcore_axis_name="s"),
    scratch_types=[
        pltpu.VMEM_SHARED((N_PAD, D_FEAT), jnp.float32),
        pltpu.VMEM((CB, K), jnp.int32),
        pltpu.VMEM((CB, K), jnp.int32),
        pltpu.VMEM((CB * K,), jnp.float32),
        pltpu.VMEM((K, D_FEAT), jnp.float32),
        pltpu.VMEM((K, D_FEAT), jnp.float32),
        pltpu.VMEM((K, D_FEAT), jnp.float32),
        pltpu.SemaphoreType.DMA,
        pltpu.SemaphoreType.DMA,
        pltpu.SemaphoreType.DMA,
        pltpu.SemaphoreType.DMA,
        pltpu.SemaphoreType.DMA,
        pltpu.SemaphoreType.DMA,
    ],
)(_sc_scatter_body)


def _finish_body(acc_ref, o_ref):
    t = acc_ref[0] + acc_ref[1]
    t = jnp.maximum(t, 0.0)
    nrm = jnp.sqrt(jnp.sum(t * t, axis=1, keepdims=True))
    o_ref[...] = t / jnp.maximum(nrm, 1e-12)


_ROWS_PER_BLK = 1024


def _finish(acc):
    return pl.pallas_call(
        _finish_body,
        grid=(N_PAD // _ROWS_PER_BLK,),
        in_specs=[pl.BlockSpec((NC, _ROWS_PER_BLK, D_FEAT),
                               lambda i: (0, i, 0))],
        out_specs=pl.BlockSpec((_ROWS_PER_BLK, D_FEAT), lambda i: (i, 0)),
        out_shape=jax.ShapeDtypeStruct((N_PAD, D_FEAT), jnp.float32),
    )(acc)


def kernel(x, edge, edge_weight):
    src = edge[0].reshape(NW, NB, CB, K)
    dst = edge[2].reshape(NW, NB, CB, K)
    ew = edge_weight.reshape(NW, NB, CB * K)
    acc = _sc_scatter(x, src, dst, ew)
    return _finish(acc)[:N_NODES]


# speedup vs baseline: 3.1854x; 1.0965x over previous
import functools

import jax
import jax.numpy as jnp
from jax import lax
from jax.experimental import pallas as pl
from jax.experimental.pallas import tpu as pltpu
from jax.experimental.pallas import tpu_sc as plsc

N_NODES = 10000
D_FEAT = 128
N_EDGES = 320000

NC = 2
NS = 16
NW = NC * NS
K = 80
EPW = N_EDGES // NW
NB = 5
CB = 25
NBUF = 3
N_PAD = 10240
RPT = N_PAD // NS


def _sc_scatter_body(x_hbm, edge_hbm, ew_hbm, acc_hbm,
                     acc_sh, src_v, dst_v, ew_v,
                     rows0, rows1, rows2,
                     g0a, g0b, g1a, g1b, g2a, g2b, s0, s1, s2):
    rows = (rows0, rows1, rows2)
    gsem = ((g0a, g0b), (g1a, g1b), (g2a, g2b))
    ssem = (s0, s1, s2)
    c = lax.axis_index("c")
    s = lax.axis_index("s")
    gid = c * NS + s

    def _zero(i, carry):
        rows0[i // 8, pl.ds((i % 8) * 16, 16)] = jnp.zeros((16,), jnp.float32)
        return carry
    lax.fori_loop(0, K * 8, _zero, 0)
    for j in range(RPT // K):
        pltpu.sync_copy(rows0, acc_sh.at[pl.ds(s * RPT + j * K, K)])
    pltpu.sync_copy(edge_hbm.at[0, gid, 0], src_v)
    pltpu.sync_copy(edge_hbm.at[2, gid, 0], dst_v)
    pltpu.sync_copy(ew_hbm.at[gid, 0], ew_v)
    plsc.subcore_barrier()

    def _scale(buf, base, g, inner):
        w_win = ew_v[pl.ds(base + g * 16, 16)]
        for r16 in range(16):
            r = g * 16 + r16
            w16 = jnp.broadcast_to(w_win[r16], (16,))
            for cc in range(8):
                sl = pl.ds(cc * 16, 16)
                buf[r, sl] = buf[r, sl] * w16
        return inner

    def _widen_scale(buf, k):
        lax.fori_loop(0, K // 16, functools.partial(_scale, buf, k * K), 0)

    H = K // 2

    def _g_issue(buf, sems, k):
        pltpu.async_copy(x_hbm.at[src_v.at[k, pl.ds(0, H)]],
                         buf.at[pl.ds(0, H)], sems[0])
        pltpu.async_copy(x_hbm.at[src_v.at[k, pl.ds(H, H)]],
                         buf.at[pl.ds(H, H)], sems[1])

    def _g_wait(buf, sems, k):
        pltpu.make_async_copy(x_hbm.at[src_v.at[k, pl.ds(0, H)]],
                              buf.at[pl.ds(0, H)], sems[0]).wait()
        pltpu.make_async_copy(x_hbm.at[src_v.at[k, pl.ds(H, H)]],
                              buf.at[pl.ds(H, H)], sems[1]).wait()

    def _s_issue(buf, sem, k):
        pltpu.async_copy(buf, acc_sh.at[dst_v.at[k]], sem, add=True)

    def _s_wait(buf, sem, k):
        pltpu.make_async_copy(buf, acc_sh.at[dst_v.at[k]], sem).wait()

    def _block(b, carry):
        @pl.when(b > 0)
        def _stage():
            pltpu.sync_copy(edge_hbm.at[0, gid, b], src_v)
            pltpu.sync_copy(edge_hbm.at[2, gid, b], dst_v)
            pltpu.sync_copy(ew_hbm.at[gid, b], ew_v)

        for i in range(NBUF):
            _g_issue(rows[i], gsem[i], i)

        def _tri(q, c2):
            k0 = q * NBUF
            for i in range(NBUF):
                k = k0 + i
                _g_wait(rows[i], gsem[i], k)
                _widen_scale(rows[i], k)
                _s_issue(rows[i], ssem[i], k)
            for i in range(NBUF):
                k = k0 + i
                _s_wait(rows[i], ssem[i], k)
                kn = k0 + NBUF + i

                @pl.when(kn < CB)
                def _issue_next(kn=kn, i=i):
                    _g_issue(rows[i], gsem[i], kn)
            return c2
        lax.fori_loop(0, CB // NBUF, _tri, 0)

        kt = (CB // NBUF) * NBUF
        _g_wait(rows0, gsem[0], kt)
        _widen_scale(rows0, kt)
        pltpu.sync_copy(rows0, acc_sh.at[dst_v.at[kt]], add=True)
        return carry
    lax.fori_loop(0, NB, _block, 0)

    plsc.subcore_barrier()
    pltpu.sync_copy(acc_sh.at[pl.ds(s * RPT, RPT)],
                    acc_hbm.at[c, pl.ds(s * RPT, RPT)])


_sc_scatter = functools.partial(
    pl.kernel,
    out_type=jax.ShapeDtypeStruct((NC, N_PAD, D_FEAT), jnp.float32),
    mesh=plsc.VectorSubcoreMesh(core_axis_name="c", subcore_axis_name="s"),
    scratch_types=[
        pltpu.VMEM_SHARED((N_PAD, D_FEAT), jnp.float32),
        pltpu.VMEM((CB, K), jnp.int32),
        pltpu.VMEM((CB, K), jnp.int32),
        pltpu.VMEM((CB * K,), jnp.float32),
        pltpu.VMEM((K, D_FEAT), jnp.float32),
        pltpu.VMEM((K, D_FEAT), jnp.float32),
        pltpu.VMEM((K, D_FEAT), jnp.float32),
        pltpu.SemaphoreType.DMA,
        pltpu.SemaphoreType.DMA,
        pltpu.SemaphoreType.DMA,
        pltpu.SemaphoreType.DMA,
        pltpu.SemaphoreType.DMA,
        pltpu.SemaphoreType.DMA,
        pltpu.SemaphoreType.DMA,
        pltpu.SemaphoreType.DMA,
        pltpu.SemaphoreType.DMA,
    ],
)(_sc_scatter_body)


def _finish_body(acc_ref, o_ref):
    t = acc_ref[0] + acc_ref[1]
    t = jnp.maximum(t, 0.0)
    nrm = jnp.sqrt(jnp.sum(t * t, axis=1, keepdims=True))
    o_ref[...] = t / jnp.maximum(nrm, 1e-12)


_ROWS_PER_BLK = N_PAD


def _finish(acc):
    return pl.pallas_call(
        _finish_body,
        grid=(1,),
        in_specs=[pl.BlockSpec((NC, _ROWS_PER_BLK, D_FEAT),
                               lambda i: (0, i, 0))],
        out_specs=pl.BlockSpec((_ROWS_PER_BLK, D_FEAT), lambda i: (i, 0)),
        out_shape=jax.ShapeDtypeStruct((N_NODES, D_FEAT), jnp.float32),
    )(acc)


def kernel(x, edge, edge_weight):
    e = edge.reshape(3, NW, NB, CB, K)
    ew = edge_weight.reshape(NW, NB, CB * K)
    acc = _sc_scatter(x, e, ew)
    return _finish(acc)
